# baseline (speedup 1.0000x reference)
import jax
import jax.numpy as jnp
from jax import lax
from jax.experimental import pallas as pl
from jax.experimental.pallas import tpu as pltpu
from jax.experimental.pallas import tpu_sc as plsc

_EMB = 128
_BATCH = 512
_NC, _NS = 2, 16
_NW = _NC * _NS
_BPW = _BATCH // _NW
_L = 16
_NCH = _EMB // _L
_NTERM = 9
_NG = 6
_WIDTHS = (2, 3, 3, 3, 2, 3)

_IDX_SRC = ((0, 0), (0, 1),
            (1, 0), (1, 1), (1, 2),
            (2, 0), (2, 1), (2, 2),
            (3, 0), (3, 1), (3, 2),
            (4, 0), (4, 1),
            (5, 0), (5, 1), (5, 2))
_GATHERS = (
    ("cls", 0, 0, 0), ("cls", 1, 1, 0),
    ("cls", 2, 2, 1), ("cls", 3, 3, 1), ("cls", 4, 4, 1),
    ("cls", 5, 5, 2), ("cls", 7, 6, 2), ("bmp", 5, 0, 2),
    ("bmp", 7, 1, 2), ("rh", 6, 0, 2), ("rt", 6, 0, 2),
    ("cls", 10, 7, 3), ("bmp", 9, 2, 3), ("rh", 8, 1, 3),
    ("cls", 11, 8, 4), ("cls", 12, 9, 4),
    ("cls", 13, 10, 5), ("cls", 15, 11, 5), ("bmp", 13, 3, 5),
    ("bmp", 15, 4, 5), ("rh", 14, 2, 5), ("rt", 14, 1, 5),
)


def _sc_body(idx_hbm, cls_hbm, bmp_hbm, rh_hbm, rt_hbm,
             out_hbm, idx_v, cls_v, bmp_v, rh_v, rt_v, out_v,
             ssem, gsem):
    wid = lax.axis_index("s") * _NC + lax.axis_index("c")
    base = wid * _BPW

    pltpu.async_copy(idx_hbm.at[:, pl.ds(wid, 1), :], idx_v, ssem).wait()

    tabs = {"cls": (cls_hbm, cls_v), "bmp": (bmp_hbm, bmp_v),
            "rh": (rh_hbm, rh_v), "rt": (rt_hbm, rt_v)}
    groups = [[] for _ in range(_NG)]
    for tab, row, slot, grp in _GATHERS:
        hbm, dst = tabs[tab]
        groups[grp].append(pltpu.async_copy(hbm.at[idx_v.at[row, 0]],
                                            dst.at[slot], gsem.at[grp]))

    def incl(c1, o1, c2, o2):
        r = jnp.maximum(jnp.abs(c1 - c2) + o1 - o2, 0.0)
        return r * r

    def disj(c1, o1, c2, o2):
        r = jnp.maximum(jnp.abs(c1 - c2) - o1 - o2, 0.0)
        return r * r

    def run_group(grp, compute_item):
        for dsc in groups[grp]:
            dsc.wait()

        def it(i, carry):
            compute_item(i)
            return carry

        lax.fori_loop(0, _BPW, it, 0)

    def C(slot, i, lo):
        return cls_v[slot, i, pl.ds(lo, _L)]

    def O(slot, i, lo):
        return jnp.abs(cls_v[slot, i, pl.ds(_EMB + lo, _L)])

    def B(slot, i, lo):
        return bmp_v[slot, i, pl.ds(lo, _L)]

    def HC(slot, i, lo):
        return rh_v[slot, i, pl.ds(lo, _L)]

    def HO(slot, i, lo):
        return jnp.abs(rh_v[slot, i, pl.ds(_EMB + lo, _L)])

    def TC(slot, i, lo):
        return rt_v[slot, i, pl.ds(lo, _L)]

    def TO(slot, i, lo):
        return jnp.abs(rt_v[slot, i, pl.ds(_EMB + lo, _L)])

    def zeros():
        return jnp.zeros((_L,), jnp.float32)

    def g0(i):
        a = zeros()
        for j in range(_NCH):
            lo = j * _L
            a = a + incl(C(0, i, lo), O(0, i, lo), C(1, i, lo), O(1, i, lo))
        out_v[i, 0, :] = a

    def g1(i):
        aA, aB = zeros(), zeros()
        for j in range(_NCH):
            lo = j * _L
            cc, co = C(2, i, lo), O(2, i, lo)
            dc, do = C(3, i, lo), O(3, i, lo)
            lower = jnp.maximum(cc - co, dc - do)
            upper = jnp.minimum(cc + co, dc + do)
            ic = 0.5 * (lower + upper)
            io = 0.5 * (upper - lower)
            aA = aA + incl(ic, io, C(4, i, lo), O(4, i, lo))
            rlu = jnp.maximum(lower - upper, 0.0)
            aB = aB + rlu * rlu
        out_v[i, 1, :] = aA
        out_v[i, 2, :] = aB

    def g2(i):
        a3, a4 = zeros(), zeros()
        for j in range(_NCH):
            lo = j * _L
            a3 = a3 + incl(C(5, i, lo) + B(1, i, lo), O(5, i, lo),
                           HC(0, i, lo), HO(0, i, lo))
            a4 = a4 + incl(C(6, i, lo) + B(0, i, lo), O(6, i, lo),
                           TC(0, i, lo), TO(0, i, lo))
        out_v[i, 3, :] = a3
        out_v[i, 4, :] = a4

    def g3(i):
        a = zeros()
        for j in range(_NCH):
            lo = j * _L
            a = a + incl(HC(1, i, lo) - B(2, i, lo), HO(1, i, lo),
                         C(7, i, lo), O(7, i, lo))
        out_v[i, 5, :] = a

    def g4(i):
        a = zeros()
        for j in range(_NCH):
            lo = j * _L
            a = a + disj(C(8, i, lo), O(8, i, lo), C(9, i, lo), O(9, i, lo))
        out_v[i, 6, :] = a

    def g5(i):
        a7, a8 = zeros(), zeros()
        for j in range(_NCH):
            lo = j * _L
            a7 = a7 + disj(C(10, i, lo) + B(4, i, lo), O(10, i, lo),
                           HC(2, i, lo), HO(2, i, lo))
            a8 = a8 + disj(C(11, i, lo) + B(3, i, lo), O(11, i, lo),
                           TC(1, i, lo), TO(1, i, lo))
        out_v[i, 7, :] = a7
        out_v[i, 8, :] = a8

    for grp, fn in enumerate((g0, g1, g2, g3, g4, g5)):
        with jax.named_scope(f"grp{grp}"):
            run_group(grp, fn)
    with jax.named_scope("outcopy"):
        pltpu.sync_copy(out_v, out_hbm.at[pl.ds(base, _BPW)])


def _reg_body(b_ref, o_ref):
    b = b_ref[...]
    reg = 0.05 * jnp.mean(jnp.sqrt(jnp.sum(b * b, axis=1)))
    o_ref[...] = jnp.reshape(reg, (1, 1))


def _tc_body(s_ref, r_ref, o_ref):
    s = jnp.sum(s_ref[...], axis=-1)
    rt = jnp.sqrt(s)
    loss1 = jnp.mean(s[:, 0])
    loss2 = (jnp.mean(s[:, 1]) + jnp.mean(s[:, 2])
             + 2.0 * jnp.mean(rt[:, 1]) * jnp.mean(rt[:, 2]))
    loss3 = 0.25 * (jnp.mean(s[:, 3]) + jnp.mean(s[:, 4])
                    + 2.0 * jnp.mean(jnp.sqrt(s[:, 3] * s[:, 4])))
    loss4 = jnp.mean(s[:, 5])
    dloss = jnp.mean((2.0 - rt[:, 6]) ** 2)
    nloss = jnp.mean((2.0 - rt[:, 7]) ** 2) + jnp.mean((2.0 - rt[:, 8]) ** 2)
    total = loss1 + loss2 + dloss + loss3 + loss4 + nloss + r_ref[0, 0]
    o_ref[...] = jnp.reshape(total, (1, 1))


def kernel(nf1, nf2, nf3, nf4, disjoint, nf3_neg, class_embeds, bumps,
           relation_heads, relation_tails):
    sc = pl.kernel(
        _sc_body,
        out_type=jax.ShapeDtypeStruct((_BATCH, _NTERM, _L), jnp.float32),
        mesh=plsc.VectorSubcoreMesh(core_axis_name="c", subcore_axis_name="s",
                                    num_cores=_NC, num_subcores=_NS),
        scratch_types=[
            pltpu.VMEM((16, 1, _BPW), jnp.int32),
            pltpu.VMEM((12, _BPW, 2 * _EMB), jnp.float32),
            pltpu.VMEM((5, _BPW, _EMB), jnp.float32),
            pltpu.VMEM((3, _BPW, 2 * _EMB), jnp.float32),
            pltpu.VMEM((2, _BPW, 2 * _EMB), jnp.float32),
            pltpu.VMEM((_BPW, _NTERM, _L), jnp.float32),
            pltpu.SemaphoreType.DMA,
            pltpu.SemaphoreType.DMA((_NG,)),
        ],
    )
    idx = jnp.stack([
        nf1[:_BATCH, 0], nf1[:_BATCH, 1],
        nf2[:_BATCH, 0], nf2[:_BATCH, 1], nf2[:_BATCH, 2],
        nf3[:_BATCH, 0], nf3[:_BATCH, 1], nf3[:_BATCH, 2],
        nf4[:_BATCH, 0], nf4[:_BATCH, 1], nf4[:_BATCH, 2],
        disjoint[:_BATCH, 0], disjoint[:_BATCH, 1],
        nf3_neg[:_BATCH, 0], nf3_neg[:_BATCH, 1], nf3_neg[:_BATCH, 2],
    ], axis=0).astype(jnp.int32).reshape(16, _NW, _BPW)

    part = sc(idx, class_embeds, bumps, relation_heads, relation_tails)

    reg = pl.pallas_call(
        _reg_body,
        out_shape=jax.ShapeDtypeStruct((1, 1), jnp.float32),
    )(bumps)
    tot = pl.pallas_call(
        _tc_body,
        out_shape=jax.ShapeDtypeStruct((1, 1), jnp.float32),
    )(part, reg)
    return tot[0, 0]


# speedup vs baseline: 1.0482x; 1.0482x over previous
import jax
import jax.numpy as jnp
from jax import lax
from jax.experimental import pallas as pl
from jax.experimental.pallas import tpu as pltpu
from jax.experimental.pallas import tpu_sc as plsc

_EMB = 128
_BATCH = 512
_NC, _NS = 2, 16
_NW = _NC * _NS
_BPW = _BATCH // _NW
_L = 16
_NCH = _EMB // _L
_NTERM = 9
_NG = 6
_WIDTHS = (2, 3, 3, 3, 2, 3)

_IDX_SRC = ((0, 0), (0, 1),
            (1, 0), (1, 1), (1, 2),
            (2, 0), (2, 1), (2, 2),
            (3, 0), (3, 1), (3, 2),
            (4, 0), (4, 1),
            (5, 0), (5, 1), (5, 2))
_GATHERS = (
    ("cls", 0, 0, 0), ("cls", 1, 1, 0),
    ("cls", 2, 2, 1), ("cls", 3, 3, 1), ("cls", 4, 4, 1),
    ("cls", 5, 5, 2), ("cls", 7, 6, 2), ("bmp", 5, 0, 2),
    ("bmp", 7, 1, 2), ("rh", 6, 0, 2), ("rt", 6, 0, 2),
    ("cls", 10, 7, 3), ("bmp", 9, 2, 3), ("rh", 8, 1, 3),
    ("cls", 11, 8, 4), ("cls", 12, 9, 4),
    ("cls", 13, 10, 5), ("cls", 15, 11, 5), ("bmp", 13, 3, 5),
    ("bmp", 15, 4, 5), ("rh", 14, 2, 5), ("rt", 14, 1, 5),
)


def _sc_body(idx_hbm, cls_hbm, bmp_hbm, rh_hbm, rt_hbm,
             out_hbm, idx_v, cls_v, bmp_v, rh_v, rt_v, out_v,
             ssem, gsem):
    wid = lax.axis_index("s") * _NC + lax.axis_index("c")
    base = wid * _BPW

    pltpu.async_copy(idx_hbm.at[:, pl.ds(wid, 1), :], idx_v, ssem).wait()

    tabs = {"cls": (cls_hbm, cls_v), "bmp": (bmp_hbm, bmp_v),
            "rh": (rh_hbm, rh_v), "rt": (rt_hbm, rt_v)}
    groups = [[] for _ in range(_NG)]
    for tab, row, slot, grp in _GATHERS:
        hbm, dst = tabs[tab]
        groups[grp].append(pltpu.async_copy(hbm.at[idx_v.at[row, 0]],
                                            dst.at[slot], gsem.at[grp]))

    def incl(c1, o1, c2, o2):
        r = jnp.maximum(jnp.abs(c1 - c2) + o1 - o2, 0.0)
        return r * r

    def disj(c1, o1, c2, o2):
        r = jnp.maximum(jnp.abs(c1 - c2) - o1 - o2, 0.0)
        return r * r

    def run_group(grp, compute_item):
        for dsc in groups[grp]:
            dsc.wait()

        def it(i, carry):
            compute_item(i)
            return carry

        lax.fori_loop(0, _BPW, it, 0)

    def C(slot, i, lo):
        return cls_v[slot, i, pl.ds(lo, _L)]

    def O(slot, i, lo):
        return jnp.abs(cls_v[slot, i, pl.ds(_EMB + lo, _L)])

    def B(slot, i, lo):
        return bmp_v[slot, i, pl.ds(lo, _L)]

    def HC(slot, i, lo):
        return rh_v[slot, i, pl.ds(lo, _L)]

    def HO(slot, i, lo):
        return jnp.abs(rh_v[slot, i, pl.ds(_EMB + lo, _L)])

    def TC(slot, i, lo):
        return rt_v[slot, i, pl.ds(lo, _L)]

    def TO(slot, i, lo):
        return jnp.abs(rt_v[slot, i, pl.ds(_EMB + lo, _L)])

    def zeros():
        return jnp.zeros((_L,), jnp.float32)

    def g0(i):
        a = zeros()
        for j in range(_NCH):
            lo = j * _L
            a = a + incl(C(0, i, lo), O(0, i, lo), C(1, i, lo), O(1, i, lo))
        out_v[i, 0, :] = a

    def g1(i):
        aA, aB = zeros(), zeros()
        for j in range(_NCH):
            lo = j * _L
            cc, co = C(2, i, lo), O(2, i, lo)
            dc, do = C(3, i, lo), O(3, i, lo)
            lower = jnp.maximum(cc - co, dc - do)
            upper = jnp.minimum(cc + co, dc + do)
            ic = 0.5 * (lower + upper)
            io = 0.5 * (upper - lower)
            aA = aA + incl(ic, io, C(4, i, lo), O(4, i, lo))
            rlu = jnp.maximum(lower - upper, 0.0)
            aB = aB + rlu * rlu
        out_v[i, 1, :] = aA
        out_v[i, 2, :] = aB

    def g2(i):
        a3, a4 = zeros(), zeros()
        for j in range(_NCH):
            lo = j * _L
            a3 = a3 + incl(C(5, i, lo) + B(1, i, lo), O(5, i, lo),
                           HC(0, i, lo), HO(0, i, lo))
            a4 = a4 + incl(C(6, i, lo) + B(0, i, lo), O(6, i, lo),
                           TC(0, i, lo), TO(0, i, lo))
        out_v[i, 3, :] = a3
        out_v[i, 4, :] = a4

    def g3(i):
        a = zeros()
        for j in range(_NCH):
            lo = j * _L
            a = a + incl(HC(1, i, lo) - B(2, i, lo), HO(1, i, lo),
                         C(7, i, lo), O(7, i, lo))
        out_v[i, 5, :] = a

    def g4(i):
        a = zeros()
        for j in range(_NCH):
            lo = j * _L
            a = a + disj(C(8, i, lo), O(8, i, lo), C(9, i, lo), O(9, i, lo))
        out_v[i, 6, :] = a

    def g5(i):
        a7, a8 = zeros(), zeros()
        for j in range(_NCH):
            lo = j * _L
            a7 = a7 + disj(C(10, i, lo) + B(4, i, lo), O(10, i, lo),
                           HC(2, i, lo), HO(2, i, lo))
            a8 = a8 + disj(C(11, i, lo) + B(3, i, lo), O(11, i, lo),
                           TC(1, i, lo), TO(1, i, lo))
        out_v[i, 7, :] = a7
        out_v[i, 8, :] = a8

    for grp, fn in enumerate((g0, g1, g2, g3, g4, g5)):
        with jax.named_scope(f"grp{grp}"):
            run_group(grp, fn)
    with jax.named_scope("outcopy"):
        pltpu.sync_copy(out_v, out_hbm.at[pl.ds(base, _BPW)])


def _tc_body(x_ref, b_ref, m_ref, o_ref):
    x = x_ref[...]
    s = jax.lax.dot_general(x, m_ref[...], (((1,), (0,)), ((), ())),
                            preferred_element_type=jnp.float32)
    rt = jnp.sqrt(s)
    u = jnp.concatenate([
        s[:, 0:1],
        s[:, 1:2], s[:, 2:3], rt[:, 1:2], rt[:, 2:3],
        s[:, 3:4] + s[:, 4:5],
        jnp.sqrt(s[:, 3:4] * s[:, 4:5]),
        s[:, 5:6],
        (2.0 - rt[:, 6:7]) ** 2,
        (2.0 - rt[:, 7:8]) ** 2, (2.0 - rt[:, 8:9]) ** 2,
        jnp.zeros((_BATCH, 5), jnp.float32),
    ], axis=1)
    m = jax.lax.dot_general(jnp.ones((1, _BATCH), jnp.float32), u,
                            (((1,), (0,)), ((), ())),
                            preferred_element_type=jnp.float32) / _BATCH
    loss1 = m[0, 0]
    loss2 = m[0, 1] + m[0, 2] + 2.0 * m[0, 3] * m[0, 4]
    loss3 = 0.25 * (m[0, 5] + 2.0 * m[0, 6])
    loss4 = m[0, 7]
    dloss = m[0, 8]
    nloss = m[0, 9] + m[0, 10]
    b = b_ref[...]
    rs = jax.lax.dot_general(b * b, jnp.ones((_EMB, 1), jnp.float32),
                             (((1,), (0,)), ((), ())),
                             preferred_element_type=jnp.float32)
    rn = jnp.sqrt(rs)
    reg = 0.05 * (jax.lax.dot_general(jnp.ones((1, rs.shape[0]), jnp.float32),
                                      rn, (((1,), (0,)), ((), ())),
                                      preferred_element_type=jnp.float32)
                  [0, 0] / rs.shape[0])
    total = loss1 + loss2 + dloss + loss3 + loss4 + nloss + reg
    o_ref[...] = jnp.reshape(total, (1, 1))


def kernel(nf1, nf2, nf3, nf4, disjoint, nf3_neg, class_embeds, bumps,
           relation_heads, relation_tails):
    sc = pl.kernel(
        _sc_body,
        out_type=jax.ShapeDtypeStruct((_BATCH, _NTERM, _L), jnp.float32),
        mesh=plsc.VectorSubcoreMesh(core_axis_name="c", subcore_axis_name="s",
                                    num_cores=_NC, num_subcores=_NS),
        scratch_types=[
            pltpu.VMEM((16, 1, _BPW), jnp.int32),
            pltpu.VMEM((12, _BPW, 2 * _EMB), jnp.float32),
            pltpu.VMEM((5, _BPW, _EMB), jnp.float32),
            pltpu.VMEM((3, _BPW, 2 * _EMB), jnp.float32),
            pltpu.VMEM((2, _BPW, 2 * _EMB), jnp.float32),
            pltpu.VMEM((_BPW, _NTERM, _L), jnp.float32),
            pltpu.SemaphoreType.DMA,
            pltpu.SemaphoreType.DMA((_NG,)),
        ],
    )
    idx = jnp.stack([
        nf1[:_BATCH, 0], nf1[:_BATCH, 1],
        nf2[:_BATCH, 0], nf2[:_BATCH, 1], nf2[:_BATCH, 2],
        nf3[:_BATCH, 0], nf3[:_BATCH, 1], nf3[:_BATCH, 2],
        nf4[:_BATCH, 0], nf4[:_BATCH, 1], nf4[:_BATCH, 2],
        disjoint[:_BATCH, 0], disjoint[:_BATCH, 1],
        nf3_neg[:_BATCH, 0], nf3_neg[:_BATCH, 1], nf3_neg[:_BATCH, 2],
    ], axis=0).astype(jnp.int32).reshape(16, _NW, _BPW)

    part = sc(idx, class_embeds, bumps, relation_heads, relation_tails)

    onehot = (jnp.arange(_NTERM * _L, dtype=jnp.int32)[:, None] // _L
              == jnp.arange(_L, dtype=jnp.int32)[None, :]).astype(jnp.float32)
    tot = pl.pallas_call(
        _tc_body,
        out_shape=jax.ShapeDtypeStruct((1, 1), jnp.float32),
    )(part.reshape(_BATCH, _NTERM * _L), bumps, onehot)
    return tot[0, 0]
